# baseline (speedup 1.0000x reference)
import functools

import jax
import jax.numpy as jnp
from jax import lax
from jax.experimental import pallas as pl
from jax.experimental.pallas import tpu as pltpu
from jax.experimental.pallas import tpu_sc as plsc

N = 10000
E = 320000
DI = 128
DO = 64
CNTW = 16
K = 80

NC = 2
NS = 16
NW = NC * NS
EPW = E // NW
NCH = -(-EPW // K)
EPP = NCH * K
NP = 10240
RPT = NP // NS
NBUF = 8
DEPTH = 3
CR = EPW // CNTW
CPC = CR // NCH


def _sc_segment_sum(with_count):
    mesh = plsc.VectorSubcoreMesh(core_axis_name="c", subcore_axis_name="s")

    out_type = [jax.ShapeDtypeStruct((NC, NP, DO), jnp.float32)]
    scratch = [
        pltpu.VMEM((NCH, K), jnp.int32),
        pltpu.VMEM((NCH, K), jnp.int32),
        pltpu.VMEM((NBUF, K, DO), jnp.float32),
        pltpu.VMEM_SHARED((NP, DO), jnp.float32),
        pltpu.SemaphoreType.DMA((NBUF,)),
        pltpu.SemaphoreType.DMA((NBUF,)),
    ]
    if with_count:
        out_type.append(jax.ShapeDtypeStruct((NW, CR, CNTW), jnp.float32))
        scratch += [
            pltpu.VMEM((CR, CNTW), jnp.float32),
        ]

    @functools.partial(
        pl.kernel, out_type=out_type, mesh=mesh, scratch_types=scratch,
        compiler_params=pltpu.CompilerParams(use_tc_tiling_on_sc=False,
                                             needs_layout_passes=False))
    def body(*refs):
        if with_count:
            (table, srcs, dsts, z64, zc,
             out_acc, out_cnt,
             src_v, dst_v, rows, acc_sh, gsem, ssem, cntb) = refs
        else:
            (table, srcs, dsts, z64,
             out_acc,
             src_v, dst_v, rows, acc_sh, gsem, ssem) = refs

        cid = lax.axis_index("c")
        sid = lax.axis_index("s")
        tile = cid * NS + sid
        rbase = sid * RPT

        pltpu.sync_copy(srcs.at[tile], src_v)
        pltpu.sync_copy(dsts.at[tile], dst_v)
        pltpu.sync_copy(z64.at[pl.ds(rbase, RPT)],
                        acc_sh.at[pl.ds(rbase, RPT)])
        if with_count:
            pltpu.sync_copy(zc, cntb)
            ones_v = jnp.full((CNTW,), 1.0, jnp.float32)
        plsc.subcore_barrier()

        def issue_gather(j):
            b = lax.rem(j, NBUF) if isinstance(j, jax.Array) else j % NBUF
            pltpu.async_copy(table.at[src_v.at[j]], rows.at[b], gsem.at[b])

        def wait_gather(j):
            b = lax.rem(j, NBUF) if isinstance(j, jax.Array) else j % NBUF
            pltpu.make_async_copy(table.at[src_v.at[j]], rows.at[b],
                                  gsem.at[b]).wait()

        def issue_scatter(j):
            b = lax.rem(j, NBUF) if isinstance(j, jax.Array) else j % NBUF
            pltpu.async_copy(rows.at[b], acc_sh.at[dst_v.at[j]], ssem.at[b],
                             add=True)

        def wait_scatter(j):
            b = lax.rem(j, NBUF) if isinstance(j, jax.Array) else j % NBUF
            pltpu.make_async_copy(rows.at[b], acc_sh.at[dst_v.at[j]],
                                  ssem.at[b]).wait()

        for jj in range(DEPTH):
            issue_gather(jj)

        LAG = NBUF - DEPTH

        def step(j, carry):
            @pl.when(j >= LAG)
            def _():
                wait_scatter(j - LAG)

            @pl.when(j + DEPTH < NCH)
            def _():
                issue_gather(j + DEPTH)
            wait_gather(j)
            issue_scatter(j)
            if with_count:
                for t in range(CPC):
                    d = dst_v[j, pl.ds(t * CNTW, CNTW)]
                    row = lax.shift_right_logical(d, 4)
                    col = lax.bitwise_and(d, 15)
                    plsc.addupdate_scatter(cntb, [row, col], ones_v)
            return carry

        lax.fori_loop(0, NCH, step, 0)
        for js in range(NCH - LAG, NCH):
            wait_scatter(js)

        if with_count:
            pltpu.sync_copy(cntb, out_cnt.at[tile])
        plsc.subcore_barrier()

        pltpu.sync_copy(acc_sh.at[pl.ds(rbase, RPT)],
                        out_acc.at[cid, pl.ds(rbase, RPT)])

    return body


_sc_pass1 = _sc_segment_sum(with_count=True)
_sc_pass2 = _sc_segment_sum(with_count=False)

_B = 2000


def _tc_a_body(x_ref, wl_ref, wr_ref, b1_ref, p_ref, r1_ref):
    xb = x_ref[...]
    p_ref[...] = jnp.dot(xb, wl_ref[...], preferred_element_type=jnp.float32)
    r1_ref[...] = (jnp.dot(xb, wr_ref[...], preferred_element_type=jnp.float32)
                   + b1_ref[...])


def _tc_b_body(acc_ref, cnt_ref, r1_ref, w2r_ref, b2_ref, h_ref, r2_ref):
    s = acc_ref[0] + acc_ref[1]
    c = jnp.sum(cnt_ref[...], axis=1)[:, None]
    inv = 1.0 / jnp.maximum(c, 1.0)
    h = jnp.maximum(s * inv + r1_ref[...], 0.0)
    h_ref[...] = h
    r2_ref[...] = (jnp.dot(h, w2r_ref[...], preferred_element_type=jnp.float32)
                   + b2_ref[...])


def _tc_c_body(acc_ref, cnt_ref, w2l_ref, r2_ref, out_ref):
    s = acc_ref[0] + acc_ref[1]
    c = jnp.sum(cnt_ref[...], axis=1)[:, None]
    mean2 = s / jnp.maximum(c, 1.0)
    out_ref[...] = (jnp.dot(mean2, w2l_ref[...],
                            preferred_element_type=jnp.float32) + r2_ref[...])


def _row_block(d):
    return pl.BlockSpec((_B, d), lambda i: (i, 0))


def _const_block(shape):
    return pl.BlockSpec(shape, lambda i: tuple(0 for _ in shape))


_GRID = N // _B

_tc_a = pl.pallas_call(
    _tc_a_body,
    grid=(_GRID,),
    in_specs=[_row_block(DI), _const_block((DI, DO)), _const_block((DI, DO)),
              _const_block((1, DO))],
    out_specs=[_row_block(DO), _row_block(DO)],
    out_shape=[jax.ShapeDtypeStruct((N, DO), jnp.float32),
               jax.ShapeDtypeStruct((N, DO), jnp.float32)],
)

_acc_block = pl.BlockSpec((NC, _B, DO), lambda i: (0, i, 0))
_cnt_block = pl.BlockSpec((_B, NW), lambda i: (i, 0))

_tc_b = pl.pallas_call(
    _tc_b_body,
    grid=(_GRID,),
    in_specs=[_acc_block, _cnt_block, _row_block(DO), _const_block((DO, DI)),
              _const_block((1, DI))],
    out_specs=[_row_block(DO), _row_block(DI)],
    out_shape=[jax.ShapeDtypeStruct((N, DO), jnp.float32),
               jax.ShapeDtypeStruct((N, DI), jnp.float32)],
)

_tc_c = pl.pallas_call(
    _tc_c_body,
    grid=(_GRID,),
    in_specs=[_acc_block, _cnt_block, _const_block((DO, DI)), _row_block(DI)],
    out_specs=_row_block(DI),
    out_shape=jax.ShapeDtypeStruct((N, DI), jnp.float32),
)


def kernel(x, edge_index, W1_l, b1, W1_r, W2_l, b2, W2_r):
    x = x.astype(jnp.float32)
    ei = edge_index.astype(jnp.int32)
    pad = jnp.zeros((NW, EPP - EPW), jnp.int32)
    src = jnp.concatenate([ei[0].reshape(NW, EPW), pad], axis=1)
    src = src.reshape(NW, NCH, K)
    dst = jnp.concatenate([ei[1].reshape(NW, EPW), pad + N], axis=1)
    dst = dst.reshape(NW, NCH, K)
    z64 = jnp.zeros((NP, DO), jnp.float32)
    zc = jnp.zeros((CR, CNTW), jnp.float32)

    p, r1 = _tc_a(x, W1_l.T, W1_r.T, b1.reshape(1, DO))
    acc1, cnt = _sc_pass1(p, src, dst, z64, zc)
    cnt2 = cnt.reshape(NW, N).T
    h, r2 = _tc_b(acc1, cnt2, r1, W2_r.T, b2.reshape(1, DI))
    (acc2,) = _sc_pass2(h, src, dst, z64)
    out = _tc_c(acc2, cnt2, W2_l.T, r2)
    return out


# speedup vs baseline: 1.0543x; 1.0543x over previous
import functools

import jax
import jax.numpy as jnp
from jax import lax
from jax.experimental import pallas as pl
from jax.experimental.pallas import tpu as pltpu
from jax.experimental.pallas import tpu_sc as plsc

N = 10000
E = 320000
DI = 128
DO = 64
CNTW = 16
K = 80

NC = 2
NS = 16
NW = NC * NS
EPW = E // NW
NCH = -(-EPW // K)
EPP = NCH * K
NP = 10240
RPT = NP // NS
NBUF = 6
DEPTH = 3
CR = EPW // CNTW
CPC = CR // NCH


def _sc_segment_sum(with_count):
    mesh = plsc.VectorSubcoreMesh(core_axis_name="c", subcore_axis_name="s")

    out_type = [jax.ShapeDtypeStruct((NC, NP, DO), jnp.float32)]
    scratch = [
        pltpu.VMEM((NCH, K), jnp.int32),
        pltpu.VMEM((NCH, K), jnp.int32),
        pltpu.VMEM((NBUF, K, DO), jnp.float32),
        pltpu.VMEM_SHARED((NP, DO), jnp.float32),
        pltpu.SemaphoreType.DMA((NBUF,)),
        pltpu.SemaphoreType.DMA((NBUF,)),
    ]
    if with_count:
        out_type.append(jax.ShapeDtypeStruct((NW, CR, CNTW), jnp.float32))
        scratch += [
            pltpu.VMEM((CR, CNTW), jnp.float32),
        ]

    @functools.partial(
        pl.kernel, out_type=out_type, mesh=mesh, scratch_types=scratch,
        compiler_params=pltpu.CompilerParams(use_tc_tiling_on_sc=False,
                                             needs_layout_passes=False))
    def body(*refs):
        if with_count:
            (table, srcs, dsts, z64, zc,
             out_acc, out_cnt,
             src_v, dst_v, rows, acc_sh, gsem, ssem, cntb) = refs
        else:
            (table, srcs, dsts, z64,
             out_acc,
             src_v, dst_v, rows, acc_sh, gsem, ssem) = refs

        cid = lax.axis_index("c")
        sid = lax.axis_index("s")
        tile = cid * NS + sid
        rbase = sid * RPT

        pltpu.sync_copy(srcs.at[tile], src_v)
        pltpu.sync_copy(dsts.at[tile], dst_v)
        pltpu.sync_copy(z64.at[pl.ds(rbase, RPT)],
                        acc_sh.at[pl.ds(rbase, RPT)])
        if with_count:
            pltpu.sync_copy(zc, cntb)
            ones_v = jnp.full((CNTW,), 1.0, jnp.float32)
        plsc.subcore_barrier()

        def issue_gather(j):
            b = lax.rem(j, NBUF) if isinstance(j, jax.Array) else j % NBUF
            pltpu.async_copy(table.at[src_v.at[j]], rows.at[b], gsem.at[b])

        def wait_gather(j):
            b = lax.rem(j, NBUF) if isinstance(j, jax.Array) else j % NBUF
            pltpu.make_async_copy(table.at[src_v.at[j]], rows.at[b],
                                  gsem.at[b]).wait()

        def issue_scatter(j):
            b = lax.rem(j, NBUF) if isinstance(j, jax.Array) else j % NBUF
            pltpu.async_copy(rows.at[b], acc_sh.at[dst_v.at[j]], ssem.at[b],
                             add=True)

        def wait_scatter(j):
            b = lax.rem(j, NBUF) if isinstance(j, jax.Array) else j % NBUF
            pltpu.make_async_copy(rows.at[b], acc_sh.at[dst_v.at[j]],
                                  ssem.at[b]).wait()

        for jj in range(DEPTH):
            issue_gather(jj)

        LAG = NBUF - DEPTH

        def step(j, carry):
            @pl.when(j >= LAG)
            def _():
                wait_scatter(j - LAG)

            @pl.when(j + DEPTH < NCH)
            def _():
                issue_gather(j + DEPTH)
            wait_gather(j)
            issue_scatter(j)
            if with_count:
                for t in range(CPC):
                    d = dst_v[j, pl.ds(t * CNTW, CNTW)]
                    row = lax.shift_right_logical(d, 4)
                    col = lax.bitwise_and(d, 15)
                    plsc.addupdate_scatter(cntb, [row, col], ones_v)
            return carry

        lax.fori_loop(0, NCH, step, 0)
        for js in range(NCH - LAG, NCH):
            wait_scatter(js)

        if with_count:
            pltpu.sync_copy(cntb, out_cnt.at[tile])
        plsc.subcore_barrier()

        pltpu.sync_copy(acc_sh.at[pl.ds(rbase, RPT)],
                        out_acc.at[cid, pl.ds(rbase, RPT)])

    return body


_sc_pass1 = _sc_segment_sum(with_count=True)
_sc_pass2 = _sc_segment_sum(with_count=False)

_B = 2000


def _tc_a_body(x_ref, wl_ref, wr_ref, b1_ref, p_ref, r1_ref):
    xb = x_ref[...]
    p_ref[...] = jnp.dot(xb, wl_ref[...], preferred_element_type=jnp.float32)
    r1_ref[...] = (jnp.dot(xb, wr_ref[...], preferred_element_type=jnp.float32)
                   + b1_ref[...])


def _tc_b_body(acc_ref, cnt_ref, r1_ref, w2r_ref, b2_ref, h_ref, r2_ref):
    s = acc_ref[0] + acc_ref[1]
    c = jnp.sum(cnt_ref[...], axis=1)[:, None]
    inv = 1.0 / jnp.maximum(c, 1.0)
    h = jnp.maximum(s * inv + r1_ref[...], 0.0)
    h_ref[...] = h
    r2_ref[...] = (jnp.dot(h, w2r_ref[...], preferred_element_type=jnp.float32)
                   + b2_ref[...])


def _tc_c_body(acc_ref, cnt_ref, w2l_ref, r2_ref, out_ref):
    s = acc_ref[0] + acc_ref[1]
    c = jnp.sum(cnt_ref[...], axis=1)[:, None]
    mean2 = s / jnp.maximum(c, 1.0)
    out_ref[...] = (jnp.dot(mean2, w2l_ref[...],
                            preferred_element_type=jnp.float32) + r2_ref[...])


def _row_block(d):
    return pl.BlockSpec((_B, d), lambda i: (i, 0))


def _const_block(shape):
    return pl.BlockSpec(shape, lambda i: tuple(0 for _ in shape))


_GRID = N // _B

_tc_a = pl.pallas_call(
    _tc_a_body,
    grid=(_GRID,),
    in_specs=[_row_block(DI), _const_block((DI, DO)), _const_block((DI, DO)),
              _const_block((1, DO))],
    out_specs=[_row_block(DO), _row_block(DO)],
    out_shape=[jax.ShapeDtypeStruct((N, DO), jnp.float32),
               jax.ShapeDtypeStruct((N, DO), jnp.float32)],
)

_acc_block = pl.BlockSpec((NC, _B, DO), lambda i: (0, i, 0))
_cnt_block = pl.BlockSpec((_B, NW), lambda i: (i, 0))

_tc_b = pl.pallas_call(
    _tc_b_body,
    grid=(_GRID,),
    in_specs=[_acc_block, _cnt_block, _row_block(DO), _const_block((DO, DI)),
              _const_block((1, DI))],
    out_specs=[_row_block(DO), _row_block(DI)],
    out_shape=[jax.ShapeDtypeStruct((N, DO), jnp.float32),
               jax.ShapeDtypeStruct((N, DI), jnp.float32)],
)

_tc_c = pl.pallas_call(
    _tc_c_body,
    grid=(_GRID,),
    in_specs=[_acc_block, _cnt_block, _const_block((DO, DI)), _row_block(DI)],
    out_specs=_row_block(DI),
    out_shape=jax.ShapeDtypeStruct((N, DI), jnp.float32),
)


def kernel(x, edge_index, W1_l, b1, W1_r, W2_l, b2, W2_r):
    x = x.astype(jnp.float32)
    ei = edge_index.astype(jnp.int32)
    pad = jnp.zeros((NW, EPP - EPW), jnp.int32)
    src = jnp.concatenate([ei[0].reshape(NW, EPW), pad], axis=1)
    src = src.reshape(NW, NCH, K)
    dst = jnp.concatenate([ei[1].reshape(NW, EPW), pad + N], axis=1)
    dst = dst.reshape(NW, NCH, K)
    z64 = jnp.zeros((NP, DO), jnp.float32)
    zc = jnp.zeros((CR, CNTW), jnp.float32)

    p, r1 = _tc_a(x, W1_l.T, W1_r.T, b1.reshape(1, DO))
    acc1, cnt = _sc_pass1(p, src, dst, z64, zc)
    cnt2 = cnt.reshape(NW, N).T
    h, r2 = _tc_b(acc1, cnt2, r1, W2_r.T, b2.reshape(1, DI))
    (acc2,) = _sc_pass2(h, src, dst, z64)
    out = _tc_c(acc2, cnt2, W2_l.T, r2)
    return out
